# Initial kernel scaffold; baseline (speedup 1.0000x reference)
import functools

import jax
import jax.numpy as jnp
from jax import lax
from jax.experimental import pallas as pl
from jax.experimental.pallas import tpu as pltpu
from jax.experimental.pallas import tpu_sc as plsc

N = 10000
E = 320000
NC = 2
NS = 16
NW = NC * NS
N_PAD = 10240
BATCH = 128
KB = 8
CHUNK = KB * BATCH
N_CHUNKS_PER_TILE = 10
E_PAD = NW * N_CHUNKS_PER_TILE * CHUNK
ROWS_PER_TILE = N_PAD // NS
DW = 16

_sc_mesh = plsc.VectorSubcoreMesh(core_axis_name="c", subcore_axis_name="s")


@functools.partial(
    pl.kernel,
    out_type=jax.ShapeDtypeStruct((NC, N_PAD, DW), jnp.float32),
    mesh=_sc_mesh,
    scratch_types=[
        pltpu.VMEM((KB, BATCH), jnp.int32),
        pltpu.VMEM((BATCH, DW), jnp.float32),
        pltpu.VMEM_SHARED((N_PAD, DW), jnp.float32),
    ],
)
def _sc_degree(colidx_hbm, ones_hbm, zeros_hbm, out_hbm, colv, onesv, acc_sh):
    c = lax.axis_index("c")
    s = lax.axis_index("s")
    w = c * NS + s
    pltpu.sync_copy(ones_hbm, onesv)
    pltpu.sync_copy(
        zeros_hbm.at[pl.ds(s * ROWS_PER_TILE, ROWS_PER_TILE)],
        acc_sh.at[pl.ds(s * ROWS_PER_TILE, ROWS_PER_TILE)],
    )
    plsc.subcore_barrier()

    def body(i, carry):
        chunk = w * N_CHUNKS_PER_TILE + i
        pltpu.sync_copy(colidx_hbm.at[chunk], colv)
        for j in range(KB):
            pltpu.sync_copy(onesv, acc_sh.at[colv.at[j]], add=True)
        return carry

    lax.fori_loop(0, N_CHUNKS_PER_TILE, body, 0)
    plsc.subcore_barrier()
    pltpu.sync_copy(
        acc_sh.at[pl.ds(s * ROWS_PER_TILE, ROWS_PER_TILE)],
        out_hbm.at[c, pl.ds(s * ROWS_PER_TILE, ROWS_PER_TILE)],
    )


@functools.partial(
    pl.kernel,
    out_type=jax.ShapeDtypeStruct((NC, N_PAD, 64), jnp.float32),
    mesh=_sc_mesh,
    scratch_types=[
        pltpu.VMEM((KB, BATCH), jnp.int32),
        pltpu.VMEM((KB, BATCH), jnp.int32),
        pltpu.VMEM((KB, BATCH, 64), jnp.float32),
        pltpu.VMEM_SHARED((N_PAD, 64), jnp.float32),
        pltpu.SemaphoreType.DMA,
    ],
)
def _sc_edge_scatter(g_hbm, rowidx_hbm, colidx_hbm, zeros_hbm, out_hbm,
                     rowv, colv, rowsv, acc_sh, sem):
    c = lax.axis_index("c")
    s = lax.axis_index("s")
    w = c * NS + s
    pltpu.sync_copy(
        zeros_hbm.at[pl.ds(s * ROWS_PER_TILE, ROWS_PER_TILE)],
        acc_sh.at[pl.ds(s * ROWS_PER_TILE, ROWS_PER_TILE)],
    )
    plsc.subcore_barrier()

    def body(i, carry):
        chunk = w * N_CHUNKS_PER_TILE + i
        pltpu.sync_copy(rowidx_hbm.at[chunk], rowv)
        pltpu.sync_copy(colidx_hbm.at[chunk], colv)
        copies = [
            pltpu.async_copy(g_hbm.at[rowv.at[j]], rowsv.at[j], sem)
            for j in range(KB)
        ]
        for cp in copies:
            cp.wait()
        for j in range(KB):
            pltpu.sync_copy(rowsv.at[j], acc_sh.at[colv.at[j]], add=True)
        return carry

    lax.fori_loop(0, N_CHUNKS_PER_TILE, body, 0)
    plsc.subcore_barrier()
    pltpu.sync_copy(
        acc_sh.at[pl.ds(s * ROWS_PER_TILE, ROWS_PER_TILE)],
        out_hbm.at[c, pl.ds(s * ROWS_PER_TILE, ROWS_PER_TILE)],
    )


def _tc_mm_body(x_ref, w_ref, o_ref):
    o_ref[...] = jnp.dot(x_ref[...], w_ref[...],
                         preferred_element_type=jnp.float32)


def _tc_mm(x, w):
    return pl.pallas_call(
        _tc_mm_body,
        out_shape=jax.ShapeDtypeStruct((x.shape[0], w.shape[1]), jnp.float32),
    )(x, w)


def _tc_norm_body(degp_ref, h_ref, g_ref, dinv_ref):
    deg = degp_ref[0, :N, 0:1] + degp_ref[1, :N, 0:1] + 1.0
    dinv = lax.rsqrt(deg)
    dinv_ref[...] = dinv
    g_ref[...] = h_ref[...] * dinv


def _tc_norm(degp, h):
    return pl.pallas_call(
        _tc_norm_body,
        out_shape=(
            jax.ShapeDtypeStruct((N, 64), jnp.float32),
            jax.ShapeDtypeStruct((N, 1), jnp.float32),
        ),
    )(degp, h)


def _tc_mid_body(sp_ref, g1_ref, dinv_ref, b1_ref, w2_ref, g2_ref):
    s = sp_ref[0, :N] + sp_ref[1, :N]
    dinv = dinv_ref[...]
    z = dinv * (s + g1_ref[...]) + b1_ref[...]
    z = jnp.maximum(z, 0.0)
    h2 = jnp.dot(z, w2_ref[...], preferred_element_type=jnp.float32)
    g2_ref[...] = h2 * dinv


def _tc_mid(sp, g1, dinv, b1, w2):
    return pl.pallas_call(
        _tc_mid_body,
        out_shape=jax.ShapeDtypeStruct((N, 64), jnp.float32),
    )(sp, g1, dinv, b1, w2)


def _tc_final_body(sp_ref, g2_ref, dinv_ref, b2_ref, o_ref):
    s = sp_ref[0, :N] + sp_ref[1, :N]
    o_ref[...] = dinv_ref[...] * (s + g2_ref[...]) + b2_ref[...]


def _tc_final(sp, g2, dinv, b2):
    return pl.pallas_call(
        _tc_final_body,
        out_shape=jax.ShapeDtypeStruct((N, 64), jnp.float32),
    )(sp, g2, dinv, b2)


def kernel(data, edge_idx, W1, b1, W2, b2):
    row = edge_idx[0].astype(jnp.int32)
    col = edge_idx[1].astype(jnp.int32)
    pad = E_PAD - E
    row_p = jnp.concatenate([row, jnp.zeros((pad,), jnp.int32)])
    col_p = jnp.concatenate([col, jnp.full((pad,), N, jnp.int32)])
    rowidx = row_p.reshape(NW * N_CHUNKS_PER_TILE, KB, BATCH)
    colidx = col_p.reshape(NW * N_CHUNKS_PER_TILE, KB, BATCH)

    ones_pay = jnp.ones((BATCH, DW), jnp.float32)
    zeros_deg = jnp.zeros((N_PAD, DW), jnp.float32)
    zeros_acc = jnp.zeros((N_PAD, 64), jnp.float32)

    degp = _sc_degree(colidx, ones_pay, zeros_deg)
    h1 = _tc_mm(data, W1)
    g1, dinv = _tc_norm(degp, h1)
    s1p = _sc_edge_scatter(g1, rowidx, colidx, zeros_acc)
    g2 = _tc_mid(s1p, g1, dinv, b1.reshape(1, 64), W2)
    s2p = _sc_edge_scatter(g2, rowidx, colidx, zeros_acc)
    out = _tc_final(s2p, g2, dinv, b2.reshape(1, 64))
    return out


# speedup vs baseline: 15.1221x; 15.1221x over previous
import functools

import jax
import jax.numpy as jnp
from jax import lax
from jax.experimental import pallas as pl
from jax.experimental.pallas import tpu as pltpu
from jax.experimental.pallas import tpu_sc as plsc

N = 10000
E = 320000
NC = 2
NS = 16
NW = NC * NS
N_PAD = 10240
BATCH = 128
KB = 8
CHUNK = KB * BATCH
N_CHUNKS_PER_TILE = 10
E_PAD = NW * N_CHUNKS_PER_TILE * CHUNK
ROWS_PER_TILE = N_PAD // NS
DW = 16

_sc_mesh = plsc.VectorSubcoreMesh(core_axis_name="c", subcore_axis_name="s")
_sc_params = pltpu.CompilerParams(use_tc_tiling_on_sc=False)


@functools.partial(
    pl.kernel,
    out_type=jax.ShapeDtypeStruct((NC, N_PAD, DW), jnp.float32),
    mesh=_sc_mesh,
    compiler_params=_sc_params,
    scratch_types=[
        pltpu.VMEM((KB, BATCH), jnp.int32),
        pltpu.VMEM((BATCH, DW), jnp.float32),
        pltpu.VMEM_SHARED((N_PAD, DW), jnp.float32),
    ],
)
def _sc_degree(colidx_hbm, ones_hbm, zeros_hbm, out_hbm, colv, onesv, acc_sh):
    c = lax.axis_index("c")
    s = lax.axis_index("s")
    w = c * NS + s
    pltpu.sync_copy(ones_hbm, onesv)
    pltpu.sync_copy(
        zeros_hbm.at[pl.ds(s * ROWS_PER_TILE, ROWS_PER_TILE)],
        acc_sh.at[pl.ds(s * ROWS_PER_TILE, ROWS_PER_TILE)],
    )
    plsc.subcore_barrier()

    def body(i, carry):
        chunk = w * N_CHUNKS_PER_TILE + i
        pltpu.sync_copy(colidx_hbm.at[chunk], colv)
        for j in range(KB):
            pltpu.sync_copy(onesv, acc_sh.at[colv.at[j]], add=True)
        return carry

    lax.fori_loop(0, N_CHUNKS_PER_TILE, body, 0)
    plsc.subcore_barrier()
    pltpu.sync_copy(
        acc_sh.at[pl.ds(s * ROWS_PER_TILE, ROWS_PER_TILE)],
        out_hbm.at[c, pl.ds(s * ROWS_PER_TILE, ROWS_PER_TILE)],
    )


@functools.partial(
    pl.kernel,
    out_type=jax.ShapeDtypeStruct((NC, N_PAD, 64), jnp.float32),
    mesh=_sc_mesh,
    compiler_params=_sc_params,
    scratch_types=[
        pltpu.VMEM((KB, BATCH), jnp.int32),
        pltpu.VMEM((KB, BATCH), jnp.int32),
        pltpu.VMEM((KB, BATCH, 64), jnp.float32),
        pltpu.VMEM_SHARED((N_PAD, 64), jnp.float32),
        pltpu.SemaphoreType.DMA,
    ],
)
def _sc_edge_scatter(g_hbm, rowidx_hbm, colidx_hbm, zeros_hbm, out_hbm,
                     rowv, colv, rowsv, acc_sh, sem):
    c = lax.axis_index("c")
    s = lax.axis_index("s")
    w = c * NS + s
    pltpu.sync_copy(
        zeros_hbm.at[pl.ds(s * ROWS_PER_TILE, ROWS_PER_TILE)],
        acc_sh.at[pl.ds(s * ROWS_PER_TILE, ROWS_PER_TILE)],
    )
    plsc.subcore_barrier()

    def body(i, carry):
        chunk = w * N_CHUNKS_PER_TILE + i
        pltpu.sync_copy(rowidx_hbm.at[chunk], rowv)
        pltpu.sync_copy(colidx_hbm.at[chunk], colv)
        copies = [
            pltpu.async_copy(g_hbm.at[rowv.at[j]], rowsv.at[j], sem)
            for j in range(KB)
        ]
        for cp in copies:
            cp.wait()
        for j in range(KB):
            pltpu.sync_copy(rowsv.at[j], acc_sh.at[colv.at[j]], add=True)
        return carry

    lax.fori_loop(0, N_CHUNKS_PER_TILE, body, 0)
    plsc.subcore_barrier()
    pltpu.sync_copy(
        acc_sh.at[pl.ds(s * ROWS_PER_TILE, ROWS_PER_TILE)],
        out_hbm.at[c, pl.ds(s * ROWS_PER_TILE, ROWS_PER_TILE)],
    )


def _tc_mm_body(x_ref, w_ref, o_ref):
    o_ref[...] = jnp.dot(x_ref[...], w_ref[...],
                         preferred_element_type=jnp.float32)


def _tc_mm(x, w):
    return pl.pallas_call(
        _tc_mm_body,
        out_shape=jax.ShapeDtypeStruct((x.shape[0], w.shape[1]), jnp.float32),
    )(x, w)


def _tc_norm_body(degp_ref, h_ref, g_ref, dinv_ref):
    deg = degp_ref[0, :N, 0:1] + degp_ref[1, :N, 0:1] + 1.0
    dinv = lax.rsqrt(deg)
    dinv_ref[...] = dinv
    g_ref[...] = h_ref[...] * dinv


def _tc_norm(degp, h):
    return pl.pallas_call(
        _tc_norm_body,
        out_shape=(
            jax.ShapeDtypeStruct((N, 64), jnp.float32),
            jax.ShapeDtypeStruct((N, 1), jnp.float32),
        ),
    )(degp, h)


def _tc_mid_body(sp_ref, g1_ref, dinv_ref, b1_ref, w2_ref, g2_ref):
    s = sp_ref[0, :N] + sp_ref[1, :N]
    dinv = dinv_ref[...]
    z = dinv * (s + g1_ref[...]) + b1_ref[...]
    z = jnp.maximum(z, 0.0)
    h2 = jnp.dot(z, w2_ref[...], preferred_element_type=jnp.float32)
    g2_ref[...] = h2 * dinv


def _tc_mid(sp, g1, dinv, b1, w2):
    return pl.pallas_call(
        _tc_mid_body,
        out_shape=jax.ShapeDtypeStruct((N, 64), jnp.float32),
    )(sp, g1, dinv, b1, w2)


def _tc_final_body(sp_ref, g2_ref, dinv_ref, b2_ref, o_ref):
    s = sp_ref[0, :N] + sp_ref[1, :N]
    o_ref[...] = dinv_ref[...] * (s + g2_ref[...]) + b2_ref[...]


def _tc_final(sp, g2, dinv, b2):
    return pl.pallas_call(
        _tc_final_body,
        out_shape=jax.ShapeDtypeStruct((N, 64), jnp.float32),
    )(sp, g2, dinv, b2)


def kernel(data, edge_idx, W1, b1, W2, b2):
    row = edge_idx[0].astype(jnp.int32)
    col = edge_idx[1].astype(jnp.int32)
    pad = E_PAD - E
    row_p = jnp.concatenate([row, jnp.zeros((pad,), jnp.int32)])
    col_p = jnp.concatenate([col, jnp.full((pad,), N, jnp.int32)])
    rowidx = row_p.reshape(NW * N_CHUNKS_PER_TILE, KB, BATCH)
    colidx = col_p.reshape(NW * N_CHUNKS_PER_TILE, KB, BATCH)

    ones_pay = jnp.ones((BATCH, DW), jnp.float32)
    zeros_deg = jnp.zeros((N_PAD, DW), jnp.float32)
    zeros_acc = jnp.zeros((N_PAD, 64), jnp.float32)

    degp = _sc_degree(colidx, ones_pay, zeros_deg)
    h1 = _tc_mm(data, W1)
    g1, dinv = _tc_norm(degp, h1)
    s1p = _sc_edge_scatter(g1, rowidx, colidx, zeros_acc)
    g2 = _tc_mid(s1p, g1, dinv, b1.reshape(1, 64), W2)
    s2p = _sc_edge_scatter(g2, rowidx, colidx, zeros_acc)
    out = _tc_final(s2p, g2, dinv, b2.reshape(1, 64))
    return out


# speedup vs baseline: 15.9311x; 1.0535x over previous
import functools

import jax
import jax.numpy as jnp
from jax import lax
from jax.experimental import pallas as pl
from jax.experimental.pallas import tpu as pltpu
from jax.experimental.pallas import tpu_sc as plsc

N = 10000
E = 320000
NC = 2
NS = 16
NW = NC * NS
N_PAD = 10240
BATCH = 128
KB = 8
CHUNK = KB * BATCH
N_CHUNKS_PER_TILE = 10
E_PAD = NW * N_CHUNKS_PER_TILE * CHUNK
ROWS_PER_TILE = N_PAD // NS
DW = 16

_sc_mesh = plsc.VectorSubcoreMesh(core_axis_name="c", subcore_axis_name="s")
_sc_params = pltpu.CompilerParams(use_tc_tiling_on_sc=False)


@functools.partial(
    pl.kernel,
    out_type=jax.ShapeDtypeStruct((NC, N_PAD, DW), jnp.float32),
    mesh=_sc_mesh,
    compiler_params=_sc_params,
    scratch_types=[
        pltpu.VMEM((KB, BATCH), jnp.int32),
        pltpu.VMEM((BATCH, DW), jnp.float32),
        pltpu.VMEM_SHARED((N_PAD, DW), jnp.float32),
    ],
)
def _sc_degree(colidx_hbm, ones_hbm, zeros_hbm, out_hbm, colv, onesv, acc_sh):
    c = lax.axis_index("c")
    s = lax.axis_index("s")
    w = c * NS + s
    pltpu.sync_copy(ones_hbm, onesv)
    pltpu.sync_copy(
        zeros_hbm.at[pl.ds(s * ROWS_PER_TILE, ROWS_PER_TILE)],
        acc_sh.at[pl.ds(s * ROWS_PER_TILE, ROWS_PER_TILE)],
    )
    plsc.subcore_barrier()

    def body(i, carry):
        chunk = w * N_CHUNKS_PER_TILE + i
        pltpu.sync_copy(colidx_hbm.at[chunk], colv)
        for j in range(KB):
            pltpu.sync_copy(
                onesv,
                acc_sh.at[plsc.Indices(colv.at[j], ignored_value=-1)],
                add=True,
            )
        return carry

    lax.fori_loop(0, N_CHUNKS_PER_TILE, body, 0)
    plsc.subcore_barrier()
    pltpu.sync_copy(
        acc_sh.at[pl.ds(s * ROWS_PER_TILE, ROWS_PER_TILE)],
        out_hbm.at[c, pl.ds(s * ROWS_PER_TILE, ROWS_PER_TILE)],
    )


@functools.partial(
    pl.kernel,
    out_type=jax.ShapeDtypeStruct((NC, N_PAD, 64), jnp.float32),
    mesh=_sc_mesh,
    compiler_params=_sc_params,
    scratch_types=[
        pltpu.VMEM((KB, BATCH), jnp.int32),
        pltpu.VMEM((KB, BATCH), jnp.int32),
        pltpu.VMEM((KB, BATCH, 64), jnp.float32),
        pltpu.VMEM_SHARED((N_PAD, 64), jnp.float32),
        pltpu.SemaphoreType.DMA,
    ],
)
def _sc_edge_scatter(g_hbm, rowidx_hbm, colidx_hbm, zeros_hbm, out_hbm,
                     rowv, colv, rowsv, acc_sh, sem):
    c = lax.axis_index("c")
    s = lax.axis_index("s")
    w = c * NS + s
    pltpu.sync_copy(
        zeros_hbm.at[pl.ds(s * ROWS_PER_TILE, ROWS_PER_TILE)],
        acc_sh.at[pl.ds(s * ROWS_PER_TILE, ROWS_PER_TILE)],
    )
    plsc.subcore_barrier()

    def body(i, carry):
        chunk = w * N_CHUNKS_PER_TILE + i
        pltpu.sync_copy(rowidx_hbm.at[chunk], rowv)
        pltpu.sync_copy(colidx_hbm.at[chunk], colv)
        copies = [
            pltpu.async_copy(g_hbm.at[rowv.at[j]], rowsv.at[j], sem)
            for j in range(KB)
        ]
        for cp in copies:
            cp.wait()
        for j in range(KB):
            pltpu.sync_copy(
                rowsv.at[j],
                acc_sh.at[plsc.Indices(colv.at[j], ignored_value=-1)],
                add=True,
            )
        return carry

    lax.fori_loop(0, N_CHUNKS_PER_TILE, body, 0)
    plsc.subcore_barrier()
    pltpu.sync_copy(
        acc_sh.at[pl.ds(s * ROWS_PER_TILE, ROWS_PER_TILE)],
        out_hbm.at[c, pl.ds(s * ROWS_PER_TILE, ROWS_PER_TILE)],
    )


def _tc_mm_body(x_ref, w_ref, o_ref):
    o_ref[...] = jnp.dot(x_ref[...], w_ref[...],
                         preferred_element_type=jnp.float32)


def _tc_mm(x, w):
    return pl.pallas_call(
        _tc_mm_body,
        out_shape=jax.ShapeDtypeStruct((x.shape[0], w.shape[1]), jnp.float32),
    )(x, w)


def _tc_norm_body(degp_ref, h_ref, g_ref, dinv_ref):
    deg = degp_ref[0, :N, 0:1] + degp_ref[1, :N, 0:1] + 1.0
    dinv = lax.rsqrt(deg)
    dinv_ref[...] = dinv
    g_ref[...] = h_ref[...] * dinv


def _tc_norm(degp, h):
    return pl.pallas_call(
        _tc_norm_body,
        out_shape=(
            jax.ShapeDtypeStruct((N, 64), jnp.float32),
            jax.ShapeDtypeStruct((N, 1), jnp.float32),
        ),
    )(degp, h)


def _tc_mid_body(sp_ref, g1_ref, dinv_ref, b1_ref, w2_ref, g2_ref):
    s = sp_ref[0, :N] + sp_ref[1, :N]
    dinv = dinv_ref[...]
    z = dinv * (s + g1_ref[...]) + b1_ref[...]
    z = jnp.maximum(z, 0.0)
    h2 = jnp.dot(z, w2_ref[...], preferred_element_type=jnp.float32)
    g2_ref[...] = h2 * dinv


def _tc_mid(sp, g1, dinv, b1, w2):
    return pl.pallas_call(
        _tc_mid_body,
        out_shape=jax.ShapeDtypeStruct((N, 64), jnp.float32),
    )(sp, g1, dinv, b1, w2)


def _tc_final_body(sp_ref, g2_ref, dinv_ref, b2_ref, o_ref):
    s = sp_ref[0, :N] + sp_ref[1, :N]
    o_ref[...] = dinv_ref[...] * (s + g2_ref[...]) + b2_ref[...]


def _tc_final(sp, g2, dinv, b2):
    return pl.pallas_call(
        _tc_final_body,
        out_shape=jax.ShapeDtypeStruct((N, 64), jnp.float32),
    )(sp, g2, dinv, b2)


def kernel(data, edge_idx, W1, b1, W2, b2):
    row = edge_idx[0].astype(jnp.int32)
    col = edge_idx[1].astype(jnp.int32)
    pad = E_PAD - E
    row_p = jnp.concatenate([row, jnp.zeros((pad,), jnp.int32)])
    col_p = jnp.concatenate([col, jnp.full((pad,), -1, jnp.int32)])
    rowidx = row_p.reshape(NW * N_CHUNKS_PER_TILE, KB, BATCH)
    colidx = col_p.reshape(NW * N_CHUNKS_PER_TILE, KB, BATCH)

    ones_pay = jnp.ones((BATCH, DW), jnp.float32)
    zeros_deg = jnp.zeros((N_PAD, DW), jnp.float32)
    zeros_acc = jnp.zeros((N_PAD, 64), jnp.float32)

    degp = _sc_degree(colidx, ones_pay, zeros_deg)
    h1 = _tc_mm(data, W1)
    g1, dinv = _tc_norm(degp, h1)
    s1p = _sc_edge_scatter(g1, rowidx, colidx, zeros_acc)
    g2 = _tc_mid(s1p, g1, dinv, b1.reshape(1, 64), W2)
    s2p = _sc_edge_scatter(g2, rowidx, colidx, zeros_acc)
    out = _tc_final(s2p, g2, dinv, b2.reshape(1, 64))
    return out


# speedup vs baseline: 16.4952x; 1.0354x over previous
import functools

import jax
import jax.numpy as jnp
from jax import lax
from jax.experimental import pallas as pl
from jax.experimental.pallas import tpu as pltpu
from jax.experimental.pallas import tpu_sc as plsc

N = 10000
E = 320000
NC = 2
NS = 16
NW = NC * NS
N_PAD = 10240
BATCH = 128
KB = 4
CHUNK = KB * BATCH
NCH = 20
JROWS = NCH * KB
E_PER_TILE = NCH * CHUNK
E_PAD = NW * E_PER_TILE
ROWS_PER_TILE = N_PAD // NS
DW = 16

_sc_mesh = plsc.VectorSubcoreMesh(core_axis_name="c", subcore_axis_name="s")
_sc_params = pltpu.CompilerParams(use_tc_tiling_on_sc=False)


@functools.partial(
    pl.kernel,
    out_type=jax.ShapeDtypeStruct((NC, N_PAD, DW), jnp.float32),
    mesh=_sc_mesh,
    compiler_params=_sc_params,
    scratch_types=[
        pltpu.VMEM((JROWS, BATCH), jnp.int32),
        pltpu.VMEM((BATCH, DW), jnp.float32),
        pltpu.VMEM_SHARED((N_PAD, DW), jnp.float32),
        pltpu.SemaphoreType.DMA,
    ],
)
def _sc_degree(colidx_hbm, ones_hbm, zeros_hbm, out_hbm, colv, onesv, acc_sh,
               sem):
    c = lax.axis_index("c")
    s = lax.axis_index("s")
    w = c * NS + s
    pltpu.sync_copy(colidx_hbm.at[w], colv)
    pltpu.sync_copy(ones_hbm, onesv)
    pltpu.sync_copy(
        zeros_hbm.at[pl.ds(s * ROWS_PER_TILE, ROWS_PER_TILE)],
        acc_sh.at[pl.ds(s * ROWS_PER_TILE, ROWS_PER_TILE)],
    )
    plsc.subcore_barrier()

    def body(i, carry):
        for j in range(KB):
            pltpu.async_copy(onesv, acc_sh.at[colv.at[i * KB + j]], sem,
                             add=True)
        for j in range(KB):
            pltpu.make_async_copy(
                zeros_hbm.at[pl.ds(0, BATCH)], onesv, sem).wait()
        return carry

    lax.fori_loop(0, NCH, body, 0)
    plsc.subcore_barrier()
    pltpu.sync_copy(
        acc_sh.at[pl.ds(s * ROWS_PER_TILE, ROWS_PER_TILE)],
        out_hbm.at[c, pl.ds(s * ROWS_PER_TILE, ROWS_PER_TILE)],
    )


@functools.partial(
    pl.kernel,
    out_type=jax.ShapeDtypeStruct((NC, N_PAD, 64), jnp.float32),
    mesh=_sc_mesh,
    compiler_params=_sc_params,
    scratch_types=[
        pltpu.VMEM((JROWS, BATCH), jnp.int32),
        pltpu.VMEM((JROWS, BATCH), jnp.int32),
        pltpu.VMEM((CHUNK, 64), jnp.float32),
        pltpu.VMEM((CHUNK, 64), jnp.float32),
        pltpu.VMEM_SHARED((N_PAD, 64), jnp.float32),
        pltpu.SemaphoreType.DMA,
        pltpu.SemaphoreType.DMA,
        pltpu.SemaphoreType.DMA,
        pltpu.SemaphoreType.DMA,
    ],
)
def _sc_edge_scatter(g_hbm, rowidx_hbm, colidx_hbm, zeros_hbm, out_hbm,
                     rowv, colv, buf0, buf1, acc_sh, sg0, sg1, ss0, ss1):
    c = lax.axis_index("c")
    s = lax.axis_index("s")
    w = c * NS + s
    pltpu.sync_copy(rowidx_hbm.at[w], rowv)
    pltpu.sync_copy(colidx_hbm.at[w], colv)
    pltpu.sync_copy(
        zeros_hbm.at[pl.ds(s * ROWS_PER_TILE, ROWS_PER_TILE)],
        acc_sh.at[pl.ds(s * ROWS_PER_TILE, ROWS_PER_TILE)],
    )
    plsc.subcore_barrier()

    def fire_gathers(chunk, buf, sem):
        for j in range(KB):
            pltpu.async_copy(
                g_hbm.at[rowv.at[chunk * KB + j]],
                buf.at[pl.ds(j * BATCH, BATCH)],
                sem,
            )

    def fire_scatters(chunk, buf, sem):
        for j in range(KB):
            pltpu.async_copy(
                buf.at[pl.ds(j * BATCH, BATCH)],
                acc_sh.at[colv.at[chunk * KB + j]],
                sem,
                add=True,
            )

    def wait_chunk(buf, sem):
        pltpu.make_async_copy(zeros_hbm.at[pl.ds(0, CHUNK)], buf, sem).wait()

    fire_gathers(0, buf0, sg0)
    fire_gathers(1, buf1, sg1)

    def body(k, carry):
        c0 = 2 * k
        wait_chunk(buf0, sg0)
        fire_scatters(c0, buf0, ss0)
        wait_chunk(buf1, sg1)
        fire_scatters(c0 + 1, buf1, ss1)
        wait_chunk(buf0, ss0)
        fire_gathers(c0 + 2, buf0, sg0)
        wait_chunk(buf1, ss1)
        fire_gathers(c0 + 3, buf1, sg1)
        return carry

    lax.fori_loop(0, NCH // 2 - 1, body, 0)
    wait_chunk(buf0, sg0)
    fire_scatters(NCH - 2, buf0, ss0)
    wait_chunk(buf1, sg1)
    fire_scatters(NCH - 1, buf1, ss1)
    wait_chunk(buf0, ss0)
    wait_chunk(buf1, ss1)
    plsc.subcore_barrier()
    pltpu.sync_copy(
        acc_sh.at[pl.ds(s * ROWS_PER_TILE, ROWS_PER_TILE)],
        out_hbm.at[c, pl.ds(s * ROWS_PER_TILE, ROWS_PER_TILE)],
    )


def _tc_mm_body(x_ref, w_ref, o_ref):
    o_ref[...] = jnp.dot(x_ref[...], w_ref[...],
                         preferred_element_type=jnp.float32)


def _tc_mm(x, w):
    return pl.pallas_call(
        _tc_mm_body,
        out_shape=jax.ShapeDtypeStruct((x.shape[0], w.shape[1]), jnp.float32),
    )(x, w)


def _tc_norm_body(degp_ref, h_ref, g_ref, dinv_ref):
    deg = degp_ref[0, :N, 0:1] + degp_ref[1, :N, 0:1] + 1.0
    dinv = lax.rsqrt(deg)
    dinv_ref[...] = dinv
    g_ref[...] = h_ref[...] * dinv


def _tc_norm(degp, h):
    return pl.pallas_call(
        _tc_norm_body,
        out_shape=(
            jax.ShapeDtypeStruct((N, 64), jnp.float32),
            jax.ShapeDtypeStruct((N, 1), jnp.float32),
        ),
    )(degp, h)


def _tc_mid_body(sp_ref, g1_ref, dinv_ref, b1_ref, w2_ref, g2_ref):
    s = sp_ref[0, :N] + sp_ref[1, :N]
    dinv = dinv_ref[...]
    z = dinv * (s + g1_ref[...]) + b1_ref[...]
    z = jnp.maximum(z, 0.0)
    h2 = jnp.dot(z, w2_ref[...], preferred_element_type=jnp.float32)
    g2_ref[...] = h2 * dinv


def _tc_mid(sp, g1, dinv, b1, w2):
    return pl.pallas_call(
        _tc_mid_body,
        out_shape=jax.ShapeDtypeStruct((N, 64), jnp.float32),
    )(sp, g1, dinv, b1, w2)


def _tc_final_body(sp_ref, g2_ref, dinv_ref, b2_ref, o_ref):
    s = sp_ref[0, :N] + sp_ref[1, :N]
    o_ref[...] = dinv_ref[...] * (s + g2_ref[...]) + b2_ref[...]


def _tc_final(sp, g2, dinv, b2):
    return pl.pallas_call(
        _tc_final_body,
        out_shape=jax.ShapeDtypeStruct((N, 64), jnp.float32),
    )(sp, g2, dinv, b2)


def kernel(data, edge_idx, W1, b1, W2, b2):
    row = edge_idx[0].astype(jnp.int32)
    col = edge_idx[1].astype(jnp.int32)
    pad = E_PAD - E
    row_p = jnp.concatenate([row, jnp.zeros((pad,), jnp.int32)])
    col_p = jnp.concatenate(
        [col, N + (jnp.arange(pad, dtype=jnp.int32) % (N_PAD - N))])
    rowidx = row_p.reshape(NW, JROWS, BATCH)
    colidx = col_p.reshape(NW, JROWS, BATCH)

    ones_pay = jnp.ones((BATCH, DW), jnp.float32)
    zeros_deg = jnp.zeros((N_PAD, DW), jnp.float32)
    zeros_acc = jnp.zeros((N_PAD, 64), jnp.float32)

    degp = _sc_degree(colidx, ones_pay, zeros_deg)
    h1 = _tc_mm(data, W1)
    g1, dinv = _tc_norm(degp, h1)
    s1p = _sc_edge_scatter(g1, rowidx, colidx, zeros_acc)
    g2 = _tc_mid(s1p, g1, dinv, b1.reshape(1, 64), W2)
    s2p = _sc_edge_scatter(g2, rowidx, colidx, zeros_acc)
    out = _tc_final(s2p, g2, dinv, b2.reshape(1, 64))
    return out


# speedup vs baseline: 18.1823x; 1.1023x over previous
import functools

import jax
import jax.numpy as jnp
from jax import lax
from jax.experimental import pallas as pl
from jax.experimental.pallas import tpu as pltpu
from jax.experimental.pallas import tpu_sc as plsc

N = 10000
E = 320000
NC = 2
NS = 16
NW = NC * NS
N_PAD = 10240
BATCH = 128
KB = 2
CHUNK = KB * BATCH
T0 = 64
T1 = 16
E_PAD = NS * (T0 + T1) * CHUNK
IDX_ROWS = E_PAD // BATCH
DEG_NCH = IDX_ROWS // NW // KB
ROWS_PER_TILE = N_PAD // NS
DW = 16

_sc_mesh = plsc.VectorSubcoreMesh(core_axis_name="c", subcore_axis_name="s")
_sc_params = pltpu.CompilerParams(use_tc_tiling_on_sc=False)
_sc_params_nl = pltpu.CompilerParams(use_tc_tiling_on_sc=False,
                                     needs_layout_passes=False)


@functools.partial(
    pl.kernel,
    out_type=jax.ShapeDtypeStruct((NW, N_PAD), jnp.float32),
    mesh=_sc_mesh,
    compiler_params=_sc_params_nl,
    scratch_types=[
        pltpu.VMEM((DEG_NCH * KB, BATCH), jnp.int32),
        pltpu.VMEM((N_PAD,), jnp.float32),
    ],
)
def _sc_degree(colidx_hbm, out_hbm, colv, deg):
    c = lax.axis_index("c")
    s = lax.axis_index("s")
    w = c * NS + s
    pltpu.sync_copy(colidx_hbm.at[pl.ds(w * DEG_NCH * KB, DEG_NCH * KB)], colv)
    ones16 = jnp.ones((16,), jnp.float32)

    def zbody(i, carry):
        deg[pl.ds(i * 16, 16)] = jnp.zeros((16,), jnp.float32)
        return carry

    lax.fori_loop(0, N_PAD // 16, zbody, 0)

    def body(i, carry):
        for l in range(BATCH // 16):
            idx = colv[i, pl.ds(l * 16, 16)]
            plsc.addupdate_scatter(deg, [idx], ones16)
        return carry

    lax.fori_loop(0, DEG_NCH * KB, body, 0)
    pltpu.sync_copy(deg, out_hbm.at[w])


@functools.partial(
    pl.kernel,
    out_type=jax.ShapeDtypeStruct((NC, N, 64), jnp.float32),
    mesh=_sc_mesh,
    compiler_params=_sc_params,
    scratch_types=[
        pltpu.VMEM((T0 * KB, BATCH), jnp.int32),
        pltpu.VMEM((T0 * KB, BATCH), jnp.int32),
        pltpu.VMEM((CHUNK, 64), jnp.float32),
        pltpu.VMEM((CHUNK, 64), jnp.float32),
        pltpu.VMEM_SHARED((N_PAD, 64), jnp.float32),
        pltpu.SemaphoreType.DMA,
        pltpu.SemaphoreType.DMA,
        pltpu.SemaphoreType.DMA,
        pltpu.SemaphoreType.DMA,
    ],
)
def _sc_edge_scatter(g_hbm, rowidx_hbm, colidx_hbm, zeros_hbm, out_hbm,
                     rowv, colv, buf0, buf1, acc_sh, sg0, sg1, ss0, ss1):
    c = lax.axis_index("c")
    s = lax.axis_index("s")
    pltpu.sync_copy(
        zeros_hbm,
        acc_sh.at[pl.ds(s * ROWS_PER_TILE, ROWS_PER_TILE)],
    )

    def fire_gathers(chunk, buf, sem):
        for j in range(KB):
            pltpu.async_copy(
                g_hbm.at[rowv.at[chunk * KB + j]],
                buf.at[pl.ds(j * BATCH, BATCH)],
                sem,
            )

    def fire_scatters(chunk, buf, sem):
        for j in range(KB):
            pltpu.async_copy(
                buf.at[pl.ds(j * BATCH, BATCH)],
                acc_sh.at[colv.at[chunk * KB + j]],
                sem,
                add=True,
            )

    def wait_chunk(buf, sem):
        pltpu.make_async_copy(zeros_hbm.at[pl.ds(0, CHUNK)], buf, sem).wait()

    def run_pipeline(nch):
        fire_gathers(0, buf0, sg0)
        fire_gathers(1, buf1, sg1)

        def body(k, carry):
            c0 = 2 * k
            wait_chunk(buf0, sg0)
            fire_scatters(c0, buf0, ss0)
            wait_chunk(buf1, sg1)
            fire_scatters(c0 + 1, buf1, ss1)
            wait_chunk(buf0, ss0)
            fire_gathers(c0 + 2, buf0, sg0)
            wait_chunk(buf1, ss1)
            fire_gathers(c0 + 3, buf1, sg1)
            return carry

        lax.fori_loop(0, nch // 2 - 1, body, 0)
        wait_chunk(buf0, sg0)
        fire_scatters(nch - 2, buf0, ss0)
        wait_chunk(buf1, sg1)
        fire_scatters(nch - 1, buf1, ss1)
        wait_chunk(buf0, ss0)
        wait_chunk(buf1, ss1)

    @pl.when(c == 0)
    def _core0():
        base = s * T0 * KB
        pltpu.sync_copy(rowidx_hbm.at[pl.ds(base, T0 * KB)], rowv)
        pltpu.sync_copy(colidx_hbm.at[pl.ds(base, T0 * KB)], colv)
        run_pipeline(T0)

    @pl.when(c == 1)
    def _core1():
        base = NS * T0 * KB + s * T1 * KB
        pltpu.sync_copy(rowidx_hbm.at[pl.ds(base, T1 * KB)],
                        rowv.at[pl.ds(0, T1 * KB)])
        pltpu.sync_copy(colidx_hbm.at[pl.ds(base, T1 * KB)],
                        colv.at[pl.ds(0, T1 * KB)])
        run_pipeline(T1)

    plsc.subcore_barrier()
    pltpu.sync_copy(
        acc_sh.at[pl.ds(s * (N // NS), N // NS)],
        out_hbm.at[c, pl.ds(s * (N // NS), N // NS)],
    )


def _tc_mm_body(x_ref, w_ref, o_ref):
    o_ref[...] = jnp.dot(x_ref[...], w_ref[...],
                         preferred_element_type=jnp.float32)


def _tc_mm(x, w):
    return pl.pallas_call(
        _tc_mm_body,
        out_shape=jax.ShapeDtypeStruct((x.shape[0], w.shape[1]), jnp.float32),
    )(x, w)


def _tc_norm_body(degp_ref, h_ref, g_ref, dinv_ref):
    deg = jnp.sum(degp_ref[...], axis=1, keepdims=True) + 1.0
    dinv = lax.rsqrt(deg)[:N]
    dinv_ref[...] = dinv
    g_ref[...] = h_ref[...] * dinv


def _tc_norm(degp, h):
    return pl.pallas_call(
        _tc_norm_body,
        out_shape=(
            jax.ShapeDtypeStruct((N, 64), jnp.float32),
            jax.ShapeDtypeStruct((N, 1), jnp.float32),
        ),
    )(degp, h)


def _tc_mid_body(sp_ref, g1_ref, dinv_ref, b1_ref, w2_ref, g2_ref):
    s = sp_ref[0, :N] + sp_ref[1, :N]
    dinv = dinv_ref[...]
    z = dinv * (s + g1_ref[...]) + b1_ref[...]
    z = jnp.maximum(z, 0.0)
    h2 = jnp.dot(z, w2_ref[...], preferred_element_type=jnp.float32)
    g2_ref[...] = h2 * dinv


def _tc_mid(sp, g1, dinv, b1, w2):
    return pl.pallas_call(
        _tc_mid_body,
        out_shape=jax.ShapeDtypeStruct((N, 64), jnp.float32),
    )(sp, g1, dinv, b1, w2)


def _tc_final_body(sp_ref, g2_ref, dinv_ref, b2_ref, o_ref):
    s = sp_ref[0, :N] + sp_ref[1, :N]
    o_ref[...] = dinv_ref[...] * (s + g2_ref[...]) + b2_ref[...]


def _tc_final(sp, g2, dinv, b2):
    return pl.pallas_call(
        _tc_final_body,
        out_shape=jax.ShapeDtypeStruct((N, 64), jnp.float32),
    )(sp, g2, dinv, b2)


def kernel(data, edge_idx, W1, b1, W2, b2):
    row = edge_idx[0].astype(jnp.int32)
    col = edge_idx[1].astype(jnp.int32)
    pad = E_PAD - E
    row_p = jnp.concatenate([row, jnp.zeros((pad,), jnp.int32)])
    col_p = jnp.concatenate(
        [col, N + (jnp.arange(pad, dtype=jnp.int32) % (N_PAD - N))])
    rowidx = row_p.reshape(IDX_ROWS, BATCH)
    colidx = col_p.reshape(IDX_ROWS, BATCH)

    zeros_acc = jnp.zeros((ROWS_PER_TILE, 64), jnp.float32)

    degp = _sc_degree(colidx)
    h1 = _tc_mm(data, W1)
    g1, dinv = _tc_norm(degp.T, h1)
    s1p = _sc_edge_scatter(g1, rowidx, colidx, zeros_acc)
    g2 = _tc_mid(s1p, g1, dinv, b1.reshape(1, 64), W2)
    s2p = _sc_edge_scatter(g2, rowidx, colidx, zeros_acc)
    out = _tc_final(s2p, g2, dinv, b2.reshape(1, 64))
    return out


# speedup vs baseline: 36.7979x; 2.0238x over previous
import functools

import jax
import jax.numpy as jnp
from jax import lax
from jax.experimental import pallas as pl
from jax.experimental.pallas import tpu as pltpu
from jax.experimental.pallas import tpu_sc as plsc

N = 10000
E = 320000
NC = 2
NS = 16
NW = NC * NS
N_PAD = 10240
BATCH = 128
CHUNK = BATCH
NCH = 80
E_PER_TILE = NCH * CHUNK
E_PAD = NW * E_PER_TILE
IDX_ROWS = E_PAD // BATCH
JROWS = NCH
KB = 2
DEG_NCH = IDX_ROWS // NW // KB
ROWS_PER_TILE = N_PAD // NS
DW = 16

_sc_mesh = plsc.VectorSubcoreMesh(core_axis_name="c", subcore_axis_name="s")
_sc_params = pltpu.CompilerParams(use_tc_tiling_on_sc=False)
_sc_params_nl = pltpu.CompilerParams(use_tc_tiling_on_sc=False,
                                     needs_layout_passes=False)


@functools.partial(
    pl.kernel,
    out_type=jax.ShapeDtypeStruct((NW, N_PAD), jnp.float32),
    mesh=_sc_mesh,
    compiler_params=_sc_params_nl,
    scratch_types=[
        pltpu.VMEM((DEG_NCH * KB, BATCH), jnp.int32),
        pltpu.VMEM((N_PAD,), jnp.float32),
    ],
)
def _sc_degree(colidx_hbm, out_hbm, colv, deg):
    c = lax.axis_index("c")
    s = lax.axis_index("s")
    w = c * NS + s
    pltpu.sync_copy(colidx_hbm.at[pl.ds(w * DEG_NCH * KB, DEG_NCH * KB)], colv)
    ones16 = jnp.ones((16,), jnp.float32)

    def zbody(i, carry):
        deg[pl.ds(i * 16, 16)] = jnp.zeros((16,), jnp.float32)
        return carry

    lax.fori_loop(0, N_PAD // 16, zbody, 0)

    def body(i, carry):
        for l in range(BATCH // 16):
            idx = colv[i, pl.ds(l * 16, 16)]
            plsc.addupdate_scatter(deg, [idx], ones16)
        return carry

    lax.fori_loop(0, DEG_NCH * KB, body, 0)
    pltpu.sync_copy(deg, out_hbm.at[w])


@functools.partial(
    pl.kernel,
    out_type=jax.ShapeDtypeStruct((NC, N, 64), jnp.float32),
    mesh=_sc_mesh,
    compiler_params=_sc_params,
    scratch_types=[
        pltpu.VMEM((JROWS, BATCH), jnp.int32),
        pltpu.VMEM((JROWS, BATCH), jnp.int32),
        pltpu.VMEM((CHUNK, 64), jnp.float32),
        pltpu.VMEM((CHUNK, 64), jnp.float32),
        pltpu.VMEM_SHARED((N_PAD, 64), jnp.float32),
        pltpu.VMEM_SHARED((N, 64), jnp.float32),
        pltpu.SemaphoreType.DMA,
        pltpu.SemaphoreType.DMA,
        pltpu.SemaphoreType.DMA,
        pltpu.SemaphoreType.DMA,
    ],
)
def _sc_edge_scatter(g_hbm, pkidx_hbm, zeros_hbm, out_hbm,
                     rowv, colv, buf0, buf1, acc_sh, g_sh, sg0, sg1, ss0, ss1):
    c = lax.axis_index("c")
    s = lax.axis_index("s")
    w = c * NS + s
    pltpu.sync_copy(pkidx_hbm.at[pl.ds(w * JROWS, JROWS)], rowv)
    pltpu.sync_copy(
        zeros_hbm,
        acc_sh.at[pl.ds(s * ROWS_PER_TILE, ROWS_PER_TILE)],
    )
    pltpu.sync_copy(
        g_hbm.at[pl.ds(s * (N // NS), N // NS)],
        g_sh.at[pl.ds(s * (N // NS), N // NS)],
    )

    def ubody(i, carry):
        rrow = rowv.at[i]
        rcol = colv.at[i]
        for l in range(BATCH // 16):
            v = rrow[pl.ds(l * 16, 16)]
            rcol[pl.ds(l * 16, 16)] = lax.shift_right_logical(v, 14)
            rrow[pl.ds(l * 16, 16)] = lax.bitwise_and(v, 16383)
        return carry

    lax.fori_loop(0, JROWS, ubody, 0)
    plsc.subcore_barrier()

    def fire_gather(chunk, buf, sem):
        pltpu.async_copy(g_sh.at[rowv.at[chunk]], buf, sem)

    def fire_scatter(chunk, buf, sem):
        pltpu.async_copy(buf, acc_sh.at[colv.at[chunk]], sem, add=True)

    def wait_chunk(buf, sem):
        pltpu.make_async_copy(zeros_hbm.at[pl.ds(0, CHUNK)], buf, sem).wait()

    fire_gather(0, buf0, sg0)
    fire_gather(1, buf1, sg1)

    def body(k, carry):
        c0 = 2 * k
        wait_chunk(buf0, sg0)
        fire_scatter(c0, buf0, ss0)
        wait_chunk(buf1, sg1)
        fire_scatter(c0 + 1, buf1, ss1)
        wait_chunk(buf0, ss0)
        fire_gather(c0 + 2, buf0, sg0)
        wait_chunk(buf1, ss1)
        fire_gather(c0 + 3, buf1, sg1)
        return carry

    lax.fori_loop(0, NCH // 2 - 1, body, 0)
    wait_chunk(buf0, sg0)
    fire_scatter(NCH - 2, buf0, ss0)
    wait_chunk(buf1, sg1)
    fire_scatter(NCH - 1, buf1, ss1)
    wait_chunk(buf0, ss0)
    wait_chunk(buf1, ss1)
    plsc.subcore_barrier()
    pltpu.sync_copy(
        acc_sh.at[pl.ds(s * (N // NS), N // NS)],
        out_hbm.at[c, pl.ds(s * (N // NS), N // NS)],
    )


def _tc_mm_body(x_ref, w_ref, o_ref):
    o_ref[...] = jnp.dot(x_ref[...], w_ref[...],
                         preferred_element_type=jnp.float32)


def _tc_mm(x, w):
    return pl.pallas_call(
        _tc_mm_body,
        out_shape=jax.ShapeDtypeStruct((x.shape[0], w.shape[1]), jnp.float32),
    )(x, w)


def _tc_norm_body(degp_ref, h_ref, g_ref, dinv_ref):
    deg = jnp.sum(degp_ref[...], axis=1, keepdims=True) + 1.0
    dinv = lax.rsqrt(deg)[:N]
    dinv_ref[...] = dinv
    g_ref[...] = h_ref[...] * dinv


def _tc_norm(degp, h):
    return pl.pallas_call(
        _tc_norm_body,
        out_shape=(
            jax.ShapeDtypeStruct((N, 64), jnp.float32),
            jax.ShapeDtypeStruct((N, 1), jnp.float32),
        ),
    )(degp, h)


def _tc_mid_body(sp_ref, g1_ref, dinv_ref, b1_ref, w2_ref, g2_ref):
    s = sp_ref[0, :N] + sp_ref[1, :N]
    dinv = dinv_ref[...]
    z = dinv * (s + g1_ref[...]) + b1_ref[...]
    z = jnp.maximum(z, 0.0)
    h2 = jnp.dot(z, w2_ref[...], preferred_element_type=jnp.float32)
    g2_ref[...] = h2 * dinv


def _tc_mid(sp, g1, dinv, b1, w2):
    return pl.pallas_call(
        _tc_mid_body,
        out_shape=jax.ShapeDtypeStruct((N, 64), jnp.float32),
    )(sp, g1, dinv, b1, w2)


def _tc_final_body(sp_ref, g2_ref, dinv_ref, b2_ref, o_ref):
    s = sp_ref[0, :N] + sp_ref[1, :N]
    o_ref[...] = dinv_ref[...] * (s + g2_ref[...]) + b2_ref[...]


def _tc_final(sp, g2, dinv, b2):
    return pl.pallas_call(
        _tc_final_body,
        out_shape=jax.ShapeDtypeStruct((N, 64), jnp.float32),
    )(sp, g2, dinv, b2)


def kernel(data, edge_idx, W1, b1, W2, b2):
    row = edge_idx[0].astype(jnp.int32)
    col = edge_idx[1].astype(jnp.int32)
    pad = E_PAD - E
    row_p = jnp.concatenate([row, jnp.zeros((pad,), jnp.int32)])
    col_p = jnp.concatenate(
        [col, N + (jnp.arange(pad, dtype=jnp.int32) % (N_PAD - N))])
    colidx = col_p.reshape(IDX_ROWS, BATCH)
    pkidx = (row_p | (col_p << 14)).reshape(IDX_ROWS, BATCH)

    zeros_acc = jnp.zeros((ROWS_PER_TILE, 64), jnp.float32)

    degp = _sc_degree(colidx)
    h1 = _tc_mm(data, W1)
    g1, dinv = _tc_norm(degp.T, h1)
    s1p = _sc_edge_scatter(g1, pkidx, zeros_acc)
    g2 = _tc_mid(s1p, g1, dinv, b1.reshape(1, 64), W2)
    s2p = _sc_edge_scatter(g2, pkidx, zeros_acc)
    out = _tc_final(s2p, g2, dinv, b2.reshape(1, 64))
    return out
